# Initial kernel scaffold; baseline (speedup 1.0000x reference)
import jax
import jax.numpy as jnp
from jax.experimental import pallas as pl


def kernel(x, edge_index, adj_vals, W, b):
    raise NotImplementedError("write your pallas kernel here")


# speedup vs baseline: 3.2545x; 3.2545x over previous
import functools

import jax
import jax.numpy as jnp
from jax import lax
from jax.experimental import pallas as pl
from jax.experimental.pallas import tpu as pltpu
from jax.experimental.pallas import tpu_sc as plsc

NC = 2
NS = 16
NW = NC * NS
LANES = 16
CHUNK = 128


def _matmul(x, W):
    n, d_in = x.shape
    d_out = W.shape[1]
    blk = 2000
    assert n % blk == 0

    def body(x_ref, w_ref, o_ref):
        o_ref[...] = jnp.dot(x_ref[...], w_ref[...],
                             preferred_element_type=jnp.float32)

    return pl.pallas_call(
        body,
        grid=(n // blk,),
        in_specs=[pl.BlockSpec((blk, d_in), lambda i: (i, 0)),
                  pl.BlockSpec((d_in, d_out), lambda i: (0, 0))],
        out_specs=pl.BlockSpec((blk, d_out), lambda i: (i, 0)),
        out_shape=jax.ShapeDtypeStruct((n, d_out), jnp.float32),
    )(x, W)


def _combine(parts, b, n):
    npad, d = parts.shape[1], parts.shape[2]
    blk = 2000
    assert n % blk == 0

    def body(p_ref, b_ref, o_ref):
        s = p_ref[0] + p_ref[1] + b_ref[...]
        o_ref[...] = jnp.maximum(s, 0.0)

    return pl.pallas_call(
        body,
        grid=(n // blk,),
        in_specs=[pl.BlockSpec((2, blk, d), lambda i: (0, i, 0)),
                  pl.BlockSpec((1, d), lambda i: (0, 0))],
        out_specs=pl.BlockSpec((blk, d), lambda i: (i, 0)),
        out_shape=jax.ShapeDtypeStruct((n, d), jnp.float32),
    )(parts, b.reshape(1, d))


def _sc_scatter(h, srcm, dstm, adjm, npad, cpw):
    d = h.shape[1]
    rows_per_tile = npad // NS
    zcopies = rows_per_tile // CHUNK
    mesh = plsc.VectorSubcoreMesh(core_axis_name="c", subcore_axis_name="s")

    @functools.partial(
        pl.kernel,
        mesh=mesh,
        out_type=jax.ShapeDtypeStruct((NC, npad, d), jnp.float32),
        scratch_types=[
            pltpu.VMEM((cpw, CHUNK), jnp.int32),
            pltpu.VMEM((cpw, CHUNK), jnp.int32),
            pltpu.VMEM((cpw, CHUNK), jnp.float32),
            pltpu.VMEM((CHUNK, d), jnp.float32),
            pltpu.VMEM_SHARED((npad, d), jnp.float32),
            pltpu.SemaphoreType.DMA,
        ],
    )
    def body(h_hbm, src_hbm, dst_hbm, adj_hbm, out_hbm,
             src_w, dst_w, adj_w, rows_v, acc_sh, sem):
        cid = lax.axis_index("c")
        sid = lax.axis_index("s")
        wid = cid * NS + sid

        zvec = jnp.zeros((LANES,), jnp.float32)

        def zrow(r, carry):
            for j in range(d // LANES):
                rows_v[r, pl.ds(j * LANES, LANES)] = zvec
            return carry
        lax.fori_loop(0, CHUNK, zrow, 0)

        row0 = sid * rows_per_tile

        def zcp(k, carry):
            pltpu.sync_copy(rows_v, acc_sh.at[pl.ds(row0 + k * CHUNK, CHUNK)])
            return carry
        lax.fori_loop(0, zcopies, zcp, 0)

        cbase = wid * cpw
        pltpu.sync_copy(src_hbm.at[pl.ds(cbase, cpw)], src_w)
        pltpu.sync_copy(dst_hbm.at[pl.ds(cbase, cpw)], dst_w)
        pltpu.sync_copy(adj_hbm.at[pl.ds(cbase, cpw)], adj_w)

        plsc.subcore_barrier()

        def chunk_body(i, carry):
            pltpu.async_copy(h_hbm.at[src_w.at[i]], rows_v, sem).wait()

            def scale_grp(g, c2):
                av = adj_w[i, pl.ds(g * LANES, LANES)]
                for l in range(LANES):
                    ei = g * LANES + l
                    s = av[l]
                    for j in range(d // LANES):
                        sl = pl.ds(j * LANES, LANES)
                        rows_v[ei, sl] = rows_v[ei, sl] * s
                return c2
            lax.fori_loop(0, CHUNK // LANES, scale_grp, 0)

            pltpu.sync_copy(rows_v, acc_sh.at[dst_w.at[i]], add=True)
            return carry
        lax.fori_loop(0, cpw, chunk_body, 0)

        plsc.subcore_barrier()

        pltpu.sync_copy(acc_sh.at[pl.ds(row0, rows_per_tile)],
                        out_hbm.at[cid, pl.ds(row0, rows_per_tile)])

    return body(h, srcm, dstm, adjm)


def kernel(x, edge_index, adj_vals, W, b):
    n, d_in = x.shape
    d = W.shape[1]
    e = adj_vals.shape[0]

    h = _matmul(x, W)

    per_w = NW * CHUNK
    cpw = (e + per_w - 1) // per_w
    cpw = ((cpw + 7) // 8) * 8
    epad = cpw * per_w
    dst = edge_index[0]
    src = edge_index[1]
    pad = epad - e
    srcm = jnp.concatenate([src, jnp.zeros((pad,), jnp.int32)]).reshape(-1, CHUNK)
    dstm = jnp.concatenate([dst, jnp.zeros((pad,), jnp.int32)]).reshape(-1, CHUNK)
    adjm = jnp.concatenate([adj_vals, jnp.zeros((pad,), jnp.float32)]).reshape(-1, CHUNK)

    npad = ((n + NS * CHUNK - 1) // (NS * CHUNK)) * (NS * CHUNK)

    parts = _sc_scatter(h, srcm, dstm, adjm, npad, cpw)

    return _combine(parts, b, n)


# speedup vs baseline: 3.7256x; 1.1447x over previous
import functools

import jax
import jax.numpy as jnp
from jax import lax
from jax.experimental import pallas as pl
from jax.experimental.pallas import tpu as pltpu
from jax.experimental.pallas import tpu_sc as plsc

NC = 2
NS = 16
NW = NC * NS
LANES = 16
CHUNK = 128


def _matmul(x, W):
    n, d_in = x.shape
    d_out = W.shape[1]
    blk = 2000
    assert n % blk == 0

    def body(x_ref, w_ref, o_ref):
        o_ref[...] = jnp.dot(x_ref[...], w_ref[...],
                             preferred_element_type=jnp.float32)

    return pl.pallas_call(
        body,
        grid=(n // blk,),
        in_specs=[pl.BlockSpec((blk, d_in), lambda i: (i, 0)),
                  pl.BlockSpec((d_in, d_out), lambda i: (0, 0))],
        out_specs=pl.BlockSpec((blk, d_out), lambda i: (i, 0)),
        out_shape=jax.ShapeDtypeStruct((n, d_out), jnp.float32),
    )(x, W)


def _combine(parts, b, n):
    npad, d = parts.shape[1], parts.shape[2]
    blk = 2000
    assert n % blk == 0

    def body(p_ref, b_ref, o_ref):
        s = p_ref[0] + p_ref[1] + b_ref[...]
        o_ref[...] = jnp.maximum(s, 0.0)

    return pl.pallas_call(
        body,
        grid=(n // blk,),
        in_specs=[pl.BlockSpec((2, blk, d), lambda i: (0, i, 0)),
                  pl.BlockSpec((1, d), lambda i: (0, 0))],
        out_specs=pl.BlockSpec((blk, d), lambda i: (i, 0)),
        out_shape=jax.ShapeDtypeStruct((n, d), jnp.float32),
    )(parts, b.reshape(1, d))


def _sc_scatter(h, srcm, dstm, adjm, npad, cpw):
    d = h.shape[1]
    rows_per_tile = npad // NS
    zcopies = rows_per_tile // CHUNK
    S = 16
    assert cpw % S == 0
    nsup = cpw // S
    mesh = plsc.VectorSubcoreMesh(core_axis_name="c", subcore_axis_name="s")

    @functools.partial(
        pl.kernel,
        mesh=mesh,
        out_type=jax.ShapeDtypeStruct((NC, npad, d), jnp.float32),
        scratch_types=[
            pltpu.VMEM((2, S, CHUNK), jnp.int32),
            pltpu.VMEM((2, S, CHUNK), jnp.int32),
            pltpu.VMEM((2, S, CHUNK), jnp.float32),
            pltpu.VMEM((CHUNK, d), jnp.float32),
            pltpu.VMEM((CHUNK, d), jnp.float32),
            pltpu.VMEM_SHARED((npad, d), jnp.float32),
            pltpu.SemaphoreType.DMA,
            pltpu.SemaphoreType.DMA,
            pltpu.SemaphoreType.DMA,
        ],
    )
    def body(h_hbm, src_hbm, dst_hbm, adj_hbm, out_hbm,
             src_b, dst_b, adj_b, rows_a, rows_b, acc_sh, sem_a, sem_b, sem_e):
        cid = lax.axis_index("c")
        sid = lax.axis_index("s")
        wid = cid * NS + sid
        cbase = wid * cpw

        def edge_load(s, slot):
            cb = cbase + s * S
            pltpu.async_copy(src_hbm.at[pl.ds(cb, S)], src_b.at[slot], sem_e)
            pltpu.async_copy(dst_hbm.at[pl.ds(cb, S)], dst_b.at[slot], sem_e)
            pltpu.async_copy(adj_hbm.at[pl.ds(cb, S)], adj_b.at[slot], sem_e)

        def edge_wait(slot):
            pltpu.make_async_copy(src_hbm.at[pl.ds(cbase, S)], src_b.at[slot], sem_e).wait()
            pltpu.make_async_copy(dst_hbm.at[pl.ds(cbase, S)], dst_b.at[slot], sem_e).wait()
            pltpu.make_async_copy(adj_hbm.at[pl.ds(cbase, S)], adj_b.at[slot], sem_e).wait()

        edge_load(0, 0)

        zvec = jnp.zeros((LANES,), jnp.float32)

        def zrow(r, carry):
            for j in range(d // LANES):
                rows_a[r, pl.ds(j * LANES, LANES)] = zvec
            return carry
        lax.fori_loop(0, CHUNK, zrow, 0)

        row0 = sid * rows_per_tile

        def zcp(k, carry):
            pltpu.sync_copy(rows_a, acc_sh.at[pl.ds(row0 + k * CHUNK, CHUNK)])
            return carry
        lax.fori_loop(0, zcopies, zcp, 0)

        edge_wait(0)
        plsc.subcore_barrier()

        def scale_chunk(rows_v, slot, i):
            def scale_grp(g, c2):
                av = adj_b[slot, i, pl.ds(g * LANES, LANES)]
                for l in range(LANES):
                    ei = g * LANES + l
                    s = av[l]
                    for j in range(d // LANES):
                        sl = pl.ds(j * LANES, LANES)
                        rows_v[ei, sl] = rows_v[ei, sl] * s
                return c2
            lax.fori_loop(0, CHUNK // LANES, scale_grp, 0)

        def sup_body(s, carry):
            slot = lax.rem(s, 2)

            @pl.when(s + 1 < nsup)
            def _prefetch_edges():
                edge_load(s + 1, 1 - slot)

            pltpu.async_copy(h_hbm.at[src_b.at[slot, 0]], rows_a, sem_a)

            def pair_body(p, c2):
                i0 = p * 2
                i1 = i0 + 1
                pltpu.make_async_copy(h_hbm.at[src_b.at[slot, i0]], rows_a, sem_a).wait()
                pltpu.async_copy(h_hbm.at[src_b.at[slot, i1]], rows_b, sem_b)
                scale_chunk(rows_a, slot, i0)
                pltpu.sync_copy(rows_a, acc_sh.at[dst_b.at[slot, i0]], add=True)

                pltpu.make_async_copy(h_hbm.at[src_b.at[slot, i1]], rows_b, sem_b).wait()

                @pl.when(p + 1 < S // 2)
                def _start_next():
                    pltpu.async_copy(h_hbm.at[src_b.at[slot, i0 + 2]], rows_a, sem_a)
                scale_chunk(rows_b, slot, i1)
                pltpu.sync_copy(rows_b, acc_sh.at[dst_b.at[slot, i1]], add=True)
                return c2
            lax.fori_loop(0, S // 2, pair_body, 0)

            @pl.when(s + 1 < nsup)
            def _wait_edges():
                edge_wait(1 - slot)
            return carry
        lax.fori_loop(0, nsup, sup_body, 0)

        plsc.subcore_barrier()

        pltpu.sync_copy(acc_sh.at[pl.ds(row0, rows_per_tile)],
                        out_hbm.at[cid, pl.ds(row0, rows_per_tile)])

    return body(h, srcm, dstm, adjm)


def kernel(x, edge_index, adj_vals, W, b):
    n, d_in = x.shape
    d = W.shape[1]
    e = adj_vals.shape[0]

    h = _matmul(x, W)

    per_w = NW * CHUNK
    cpw = (e + per_w - 1) // per_w
    cpw = ((cpw + 15) // 16) * 16
    epad = cpw * per_w
    dst = edge_index[0]
    src = edge_index[1]
    pad = epad - e
    srcm = jnp.concatenate([src, jnp.zeros((pad,), jnp.int32)]).reshape(-1, CHUNK)
    dstm = jnp.concatenate([dst, jnp.zeros((pad,), jnp.int32)]).reshape(-1, CHUNK)
    adjm = jnp.concatenate([adj_vals, jnp.zeros((pad,), jnp.float32)]).reshape(-1, CHUNK)

    npad = ((n + NS * CHUNK - 1) // (NS * CHUNK)) * (NS * CHUNK)

    parts = _sc_scatter(h, srcm, dstm, adjm, npad, cpw)

    return _combine(parts, b, n)
